# baseline (speedup 1.0000x reference)
import jax
import jax.numpy as jnp
from jax.experimental import pallas as pl
from jax.experimental.pallas import tpu as pltpu

B = 8192
D = 768
L = 256
K = 8192

RB = 512
KB = 512
NR = B // RB
NK = K // KB
NT = NR * NK + 1
XCB = 128
NXC = D // XCB


def _dkn_body(x_ref, ax_ref, h_ref, c_ref, recon_ref, cl_ref,
              hc_buf, min_scr, h2_scr, c2h_scr):
    t = pl.program_id(0)
    k = t % NK
    slot = t % 2
    kp = (t + NK - 1) % NK
    pslot = (t + 1) % 2

    @pl.when(t == 0)
    def _init_out():
        recon_ref[...] = jnp.zeros_like(recon_ref)
        cl_ref[...] = jnp.zeros_like(cl_ref)

    h = h_ref[...]
    c = c_ref[...]
    hc = jax.lax.dot_general(h, c, (((1,), (1,)), ((), ())),
                             preferred_element_type=jnp.float32)
    hc_buf[slot] = hc

    @pl.when(t < NK)
    def _c2():
        ones = jnp.ones((1, L), jnp.bfloat16)
        c2 = jax.lax.dot_general(ones, c * c, (((1,), (1,)), ((), ())),
                                 preferred_element_type=jnp.float32)
        c2h_scr[k] = 0.5 * c2

    @pl.when(k == NK - 1)
    def _h2():
        h32 = h.astype(jnp.float32)
        h2_scr[...] = jnp.sum(h32 * h32, axis=1, keepdims=True)

    @pl.when((k < NXC) & (t < NR * NK))
    def _recon():
        d = x_ref[...] - ax_ref[...]
        dd = d * d
        ones_r = jnp.ones((1, RB), jnp.bfloat16)
        part = jax.lax.dot_general(ones_r, dd, (((1,), (0,)), ((), ())),
                                   preferred_element_type=jnp.float32)
        recon_ref[...] += jnp.sum(part)

    hcp = hc_buf[pslot]
    c2h = c2h_scr[kp]
    part = jnp.min(c2h - hcp, axis=1, keepdims=True)
    min_scr[...] = jnp.where(kp == 0, part,
                             jnp.minimum(min_scr[...], part))

    @pl.when(kp == NK - 1)
    def _cl_final():
        d2min = jnp.maximum(h2_scr[...] + 2.0 * min_scr[...], 0.0)
        s = jnp.sum(d2min)
        cl_ref[...] += jnp.where(t > 0, s, 0.0)


def kernel(x, h_x, a_x, cluster_centers):
    xb = x.astype(jnp.bfloat16)
    axb = a_x.astype(jnp.bfloat16)
    hb = h_x.astype(jnp.bfloat16)
    cb = cluster_centers.astype(jnp.bfloat16)
    recon_sum, cl_sum = pl.pallas_call(
        _dkn_body,
        grid=(NT,),
        in_specs=[
            pl.BlockSpec((RB, XCB),
                         lambda t: (jnp.minimum(t // NK, NR - 1),
                                    jnp.minimum(t % NK, NXC - 1))),
            pl.BlockSpec((RB, XCB),
                         lambda t: (jnp.minimum(t // NK, NR - 1),
                                    jnp.minimum(t % NK, NXC - 1))),
            pl.BlockSpec((RB, L),
                         lambda t: (jnp.minimum(t // NK, NR - 1), 0)),
            pl.BlockSpec((KB, L),
                         lambda t: (jnp.where(t == NR * NK, NK - 1, t % NK),
                                    0)),
        ],
        out_specs=[
            pl.BlockSpec((1, 1), lambda t: (0, 0)),
            pl.BlockSpec((1, 1), lambda t: (0, 0)),
        ],
        out_shape=[
            jax.ShapeDtypeStruct((1, 1), jnp.float32),
            jax.ShapeDtypeStruct((1, 1), jnp.float32),
        ],
        scratch_shapes=[
            pltpu.VMEM((2, RB, KB), jnp.float32),
            pltpu.VMEM((RB, 1), jnp.float32),
            pltpu.VMEM((RB, 1), jnp.float32),
            pltpu.VMEM((NK, 1, KB), jnp.float32),
        ],
    )(xb, axb, hb, cb)
    return (recon_sum[0, 0] / (B * D)) + (cl_sum[0, 0] / (B * L))


# speedup vs baseline: 1.0683x; 1.0683x over previous
import jax
import jax.numpy as jnp
from jax.experimental import pallas as pl
from jax.experimental.pallas import tpu as pltpu

B = 8192
D = 768
L = 256
K = 8192

RB = 512
KB = 512
NR = B // RB
NK = K // KB
NT = NR * NK + 1
XCB = 128
NXC = D // XCB


def _dkn_body(x_ref, ax_ref, h_ref, c_ref, recon_ref, cl_ref,
              hc_buf, min_scr, h2_scr, c2h_scr, hb_scr):
    t = pl.program_id(0)
    k = t % NK
    slot = t % 2
    kp = (t + NK - 1) % NK
    pslot = (t + 1) % 2

    @pl.when(t == 0)
    def _init_out():
        recon_ref[...] = jnp.zeros_like(recon_ref)
        cl_ref[...] = jnp.zeros_like(cl_ref)

    hcp = hc_buf[pslot]
    c2h = c2h_scr[kp]
    part = jnp.min(c2h - hcp, axis=1, keepdims=True)
    min_scr[...] = jnp.where(kp == 0, part,
                             jnp.minimum(min_scr[...], part))

    @pl.when(kp == NK - 1)
    def _cl_final():
        d2min = jnp.maximum(h2_scr[...] + 2.0 * min_scr[...], 0.0)
        s = jnp.sum(d2min)
        cl_ref[...] += jnp.where(t > 0, s, 0.0)

    c = c_ref[...].astype(jnp.bfloat16)

    @pl.when(k == 0)
    def _hcast():
        hb_scr[...] = h_ref[...].astype(jnp.bfloat16)

    @pl.when(k == NK - 1)
    def _h2():
        h32 = h_ref[...]
        h2_scr[...] = jnp.sum(h32 * h32, axis=1, keepdims=True)

    @pl.when(t < NK)
    def _c2():
        ones = jnp.ones((1, L), jnp.bfloat16)
        c2 = jax.lax.dot_general(ones, c * c, (((1,), (1,)), ((), ())),
                                 preferred_element_type=jnp.float32)
        c2h_scr[k] = 0.5 * c2

    @pl.when((k < NXC) & (t < NR * NK))
    def _recon():
        d = (x_ref[...] - ax_ref[...]).astype(jnp.bfloat16)
        dd = d * d
        ones_r = jnp.ones((1, RB), jnp.bfloat16)
        rp = jax.lax.dot_general(ones_r, dd, (((1,), (0,)), ((), ())),
                                 preferred_element_type=jnp.float32)
        recon_ref[...] += jnp.sum(rp)

    hc = jax.lax.dot_general(hb_scr[...], c, (((1,), (1,)), ((), ())),
                             preferred_element_type=jnp.float32)
    hc_buf[slot] = hc


def kernel(x, h_x, a_x, cluster_centers):
    recon_sum, cl_sum = pl.pallas_call(
        _dkn_body,
        grid=(NT,),
        in_specs=[
            pl.BlockSpec((RB, XCB),
                         lambda t: (jnp.minimum(t // NK, NR - 1),
                                    jnp.minimum(t % NK, NXC - 1))),
            pl.BlockSpec((RB, XCB),
                         lambda t: (jnp.minimum(t // NK, NR - 1),
                                    jnp.minimum(t % NK, NXC - 1))),
            pl.BlockSpec((RB, L),
                         lambda t: (jnp.minimum(t // NK, NR - 1), 0)),
            pl.BlockSpec((KB, L),
                         lambda t: (jnp.where(t == NR * NK, NK - 1, t % NK),
                                    0)),
        ],
        out_specs=[
            pl.BlockSpec((1, 1), lambda t: (0, 0)),
            pl.BlockSpec((1, 1), lambda t: (0, 0)),
        ],
        out_shape=[
            jax.ShapeDtypeStruct((1, 1), jnp.float32),
            jax.ShapeDtypeStruct((1, 1), jnp.float32),
        ],
        scratch_shapes=[
            pltpu.VMEM((2, RB, KB), jnp.float32),
            pltpu.VMEM((RB, 1), jnp.float32),
            pltpu.VMEM((RB, 1), jnp.float32),
            pltpu.VMEM((NK, 1, KB), jnp.float32),
            pltpu.VMEM((RB, L), jnp.bfloat16),
        ],
    )(x, a_x, h_x, cluster_centers)
    return (recon_sum[0, 0] / (B * D)) + (cl_sum[0, 0] / (B * L))


# speedup vs baseline: 1.3349x; 1.2496x over previous
import jax
import jax.numpy as jnp
from jax.experimental import pallas as pl
from jax.experimental.pallas import tpu as pltpu

B = 8192
D = 768
L = 256
K = 8192

RB = 512
KB = 512
NR = B // RB
NK = K // KB
NS = NR * NK
NT = NS + 1
XRB = B // NS


def _dkn_body(x_ref, ax_ref, hm_ref, hsq_ref, c_ref, recon_ref, cl_ref,
              hc_buf, c2_buf, min_scr, accx, acch, msum):
    t = pl.program_id(0)
    k = t % NK
    slot = t % 2
    kp = (t + NK - 1) % NK
    pslot = (t + 1) % 2

    keep = jnp.where(t > 0, 1.0, 0.0)
    gx = jnp.where(t < NS, 1.0, 0.0)
    mgate = jnp.where((kp == NK - 1) & (t > 0), 1.0, 0.0)

    cmat = c_ref[...]
    hmat = hm_ref[...]
    hcT = jax.lax.dot_general(cmat, hmat, (((1,), (1,)), ((), ())),
                              preferred_element_type=jnp.float32)

    hcp = hc_buf[pslot]
    c2p = c2_buf[pslot]
    partT = jnp.min(c2p - hcp, axis=0, keepdims=True)
    mprev = jnp.minimum(min_scr[...], partT)
    min_scr[...] = jnp.where(kp == 0, partT, mprev)
    msum[...] = (jnp.where(keep > 0, msum[...], 0.0)
                 + jnp.where(mgate > 0, jnp.sum(min_scr[...]), 0.0))

    d = x_ref[...] - ax_ref[...]
    accx[...] = jnp.where(keep > 0, accx[...], 0.0) + gx * (d * d)
    hs = hsq_ref[...]
    acch[...] = jnp.where(keep > 0, acch[...], 0.0) + gx * (hs * hs)

    c32 = cmat.astype(jnp.float32)
    c2_buf[slot] = 0.5 * jnp.sum(c32 * c32, axis=1, keepdims=True)

    recon_ref[...] = jnp.sum(accx[...]).reshape(1, 1)
    cl_ref[...] = jnp.sum(acch[...]) + 2.0 * msum[...]

    hc_buf[slot] = hcT


def kernel(x, h_x, a_x, cluster_centers):
    hb = h_x.astype(jnp.bfloat16)
    cb = cluster_centers.astype(jnp.bfloat16)
    recon_sum, cl_sum = pl.pallas_call(
        _dkn_body,
        grid=(NT,),
        in_specs=[
            pl.BlockSpec((XRB, D), lambda t: (jnp.minimum(t, NS - 1), 0)),
            pl.BlockSpec((XRB, D), lambda t: (jnp.minimum(t, NS - 1), 0)),
            pl.BlockSpec((RB, L),
                         lambda t: (jnp.minimum(t // NK, NR - 1), 0)),
            pl.BlockSpec((XRB, L), lambda t: (jnp.minimum(t, NS - 1), 0)),
            pl.BlockSpec((KB, L),
                         lambda t: (jnp.where(t == NS, NK - 1, t % NK), 0)),
        ],
        out_specs=[
            pl.BlockSpec((1, 1), lambda t: (0, 0)),
            pl.BlockSpec((1, 1), lambda t: (0, 0)),
        ],
        out_shape=[
            jax.ShapeDtypeStruct((1, 1), jnp.float32),
            jax.ShapeDtypeStruct((1, 1), jnp.float32),
        ],
        scratch_shapes=[
            pltpu.VMEM((2, KB, RB), jnp.float32),
            pltpu.VMEM((2, KB, 1), jnp.float32),
            pltpu.VMEM((1, RB), jnp.float32),
            pltpu.VMEM((XRB, D), jnp.float32),
            pltpu.VMEM((XRB, L), jnp.float32),
            pltpu.VMEM((1, 1), jnp.float32),
        ],
    )(x, a_x, hb, h_x, cb)
    return (recon_sum[0, 0] / (B * D)) + (cl_sum[0, 0] / (B * L))


# speedup vs baseline: 1.8864x; 1.4131x over previous
import jax
import jax.numpy as jnp
from jax.experimental import pallas as pl
from jax.experimental.pallas import tpu as pltpu

B = 8192
D = 768
L = 256
K = 8192

RB = 1024
KB = 512
NR = B // RB
NK = K // KB
NS = NR * NK
NT = NS + 1
XRB = B // NS


def _dkn_body(x_ref, ax_ref, hm_ref, hsq_ref, c_ref, recon_ref, cl_ref,
              hc_buf, c2_buf, min_scr, accx, acch, msum):
    t = pl.program_id(0)
    k = t % NK
    slot = t % 2
    kp = (t + NK - 1) % NK
    pslot = (t + 1) % 2

    keep = jnp.where(t > 0, 1.0, 0.0)
    gx = jnp.where(t < NS, 1.0, 0.0)
    mgate = jnp.where((kp == NK - 1) & (t > 0), 1.0, 0.0)

    cmat = c_ref[...]
    hmat = hm_ref[...]
    hcT = jax.lax.dot_general(cmat, hmat, (((1,), (1,)), ((), ())),
                              preferred_element_type=jnp.float32)

    hcp = hc_buf[pslot]
    c2p = c2_buf[pslot]
    partT = jnp.min(c2p - hcp, axis=0, keepdims=True)
    mprev = jnp.minimum(min_scr[...], partT)
    min_scr[...] = jnp.where(kp == 0, partT, mprev)
    msum[...] = (jnp.where(keep > 0, msum[...], 0.0)
                 + jnp.where(mgate > 0, jnp.sum(min_scr[...]), 0.0))

    d = x_ref[...] - ax_ref[...]
    accx[...] = jnp.where(keep > 0, accx[...], 0.0) + gx * (d * d)
    hs = hsq_ref[...]
    acch[...] = jnp.where(keep > 0, acch[...], 0.0) + gx * (hs * hs)

    c32 = cmat.astype(jnp.float32)
    c2_buf[slot] = 0.5 * jnp.sum(c32 * c32, axis=1, keepdims=True)

    recon_ref[...] = jnp.sum(accx[...]).reshape(1, 1)
    cl_ref[...] = jnp.sum(acch[...]) + 2.0 * msum[...]

    hc_buf[slot] = hcT


def kernel(x, h_x, a_x, cluster_centers):
    hb = h_x.astype(jnp.bfloat16)
    cb = cluster_centers.astype(jnp.bfloat16)
    recon_sum, cl_sum = pl.pallas_call(
        _dkn_body,
        grid=(NT,),
        in_specs=[
            pl.BlockSpec((XRB, D), lambda t: (jnp.minimum(t, NS - 1), 0)),
            pl.BlockSpec((XRB, D), lambda t: (jnp.minimum(t, NS - 1), 0)),
            pl.BlockSpec((RB, L),
                         lambda t: (jnp.minimum(t // NK, NR - 1), 0)),
            pl.BlockSpec((XRB, L), lambda t: (jnp.minimum(t, NS - 1), 0)),
            pl.BlockSpec((KB, L),
                         lambda t: (jnp.where(t == NS, NK - 1, t % NK), 0)),
        ],
        out_specs=[
            pl.BlockSpec((1, 1), lambda t: (0, 0)),
            pl.BlockSpec((1, 1), lambda t: (0, 0)),
        ],
        out_shape=[
            jax.ShapeDtypeStruct((1, 1), jnp.float32),
            jax.ShapeDtypeStruct((1, 1), jnp.float32),
        ],
        scratch_shapes=[
            pltpu.VMEM((2, KB, RB), jnp.float32),
            pltpu.VMEM((2, KB, 1), jnp.float32),
            pltpu.VMEM((1, RB), jnp.float32),
            pltpu.VMEM((XRB, D), jnp.float32),
            pltpu.VMEM((XRB, L), jnp.float32),
            pltpu.VMEM((1, 1), jnp.float32),
        ],
    )(x, a_x, hb, h_x, cb)
    return (recon_sum[0, 0] / (B * D)) + (cl_sum[0, 0] / (B * L))


# speedup vs baseline: 2.1554x; 1.1426x over previous
import jax
import jax.numpy as jnp
from jax.experimental import pallas as pl
from jax.experimental.pallas import tpu as pltpu

B = 8192
D = 768
L = 256
K = 8192

RB = 1024
KB = 1024
NR = B // RB
NK = K // KB
NS = NR * NK
NT = NS + 1
XRB = B // NS


def _dkn_body(x_ref, ax_ref, hm_ref, hsq_ref, c_ref, recon_ref, cl_ref,
              hc_buf, c2_buf, min_scr, accx, acch, msum):
    t = pl.program_id(0)
    k = t % NK
    slot = t % 2
    kp = (t + NK - 1) % NK
    pslot = (t + 1) % 2

    keep = jnp.where(t > 0, 1.0, 0.0)
    gx = jnp.where(t < NS, 1.0, 0.0)
    mgate = jnp.where((kp == NK - 1) & (t > 0), 1.0, 0.0)

    cmat = c_ref[...]
    hmat = hm_ref[...]
    hcT = jax.lax.dot_general(cmat, hmat, (((1,), (1,)), ((), ())),
                              preferred_element_type=jnp.float32)

    hcp = hc_buf[pslot]
    c2p = c2_buf[pslot]
    partT = jnp.min(c2p - hcp, axis=0, keepdims=True)
    mprev = jnp.minimum(min_scr[...], partT)
    min_scr[...] = jnp.where(kp == 0, partT, mprev)
    msum[...] = (jnp.where(keep > 0, msum[...], 0.0)
                 + jnp.where(mgate > 0, jnp.sum(min_scr[...]), 0.0))

    d = x_ref[...] - ax_ref[...]
    accx[...] = jnp.where(keep > 0, accx[...], 0.0) + gx * (d * d)
    hs = hsq_ref[...]
    acch[...] = jnp.where(keep > 0, acch[...], 0.0) + gx * (hs * hs)

    c32 = cmat.astype(jnp.float32)
    c2_buf[slot] = 0.5 * jnp.sum(c32 * c32, axis=1, keepdims=True)

    recon_ref[...] = jnp.sum(accx[...]).reshape(1, 1)
    cl_ref[...] = jnp.sum(acch[...]) + 2.0 * msum[...]

    hc_buf[slot] = hcT


def kernel(x, h_x, a_x, cluster_centers):
    hb = h_x.astype(jnp.bfloat16)
    cb = cluster_centers.astype(jnp.bfloat16)
    recon_sum, cl_sum = pl.pallas_call(
        _dkn_body,
        grid=(NT,),
        in_specs=[
            pl.BlockSpec((XRB, D), lambda t: (jnp.minimum(t, NS - 1), 0)),
            pl.BlockSpec((XRB, D), lambda t: (jnp.minimum(t, NS - 1), 0)),
            pl.BlockSpec((RB, L),
                         lambda t: (jnp.minimum(t // NK, NR - 1), 0)),
            pl.BlockSpec((XRB, L), lambda t: (jnp.minimum(t, NS - 1), 0)),
            pl.BlockSpec((KB, L),
                         lambda t: (jnp.where(t == NS, NK - 1, t % NK), 0)),
        ],
        out_specs=[
            pl.BlockSpec((1, 1), lambda t: (0, 0)),
            pl.BlockSpec((1, 1), lambda t: (0, 0)),
        ],
        out_shape=[
            jax.ShapeDtypeStruct((1, 1), jnp.float32),
            jax.ShapeDtypeStruct((1, 1), jnp.float32),
        ],
        scratch_shapes=[
            pltpu.VMEM((2, KB, RB), jnp.float32),
            pltpu.VMEM((2, KB, 1), jnp.float32),
            pltpu.VMEM((1, RB), jnp.float32),
            pltpu.VMEM((XRB, D), jnp.float32),
            pltpu.VMEM((XRB, L), jnp.float32),
            pltpu.VMEM((1, 1), jnp.float32),
        ],
    )(x, a_x, hb, h_x, cb)
    return (recon_sum[0, 0] / (B * D)) + (cl_sum[0, 0] / (B * L))
